# Initial kernel scaffold; baseline (speedup 1.0000x reference)
import functools

import jax
import jax.numpy as jnp
from jax import lax
from jax.experimental import pallas as pl
from jax.experimental.pallas import tpu as pltpu
from jax.experimental.pallas import tpu_sc as plsc

N_NODES = 10000
D = 128
NC = 2
NS = 16
NW = NC * NS
E = 320000
EPW = E // NW
C = 128
NCH = -(-EPW // C)
EPT = NCH * C
NPAD = 10016
RPT = NPAD // NS


def _make_agg(with_deg):
  mesh = plsc.VectorSubcoreMesh(core_axis_name="c", subcore_axis_name="s")
  out_type = [jax.ShapeDtypeStruct((NC, NS, RPT, D), jnp.float32)]
  scratch = [
      pltpu.VMEM((C,), jnp.int32),
      pltpu.VMEM((C,), jnp.int32),
      pltpu.VMEM((C, D), jnp.float32),
      pltpu.VMEM_SHARED((NPAD, D), jnp.float32),
      pltpu.SemaphoreType.DMA,
  ]
  if with_deg:
    out_type.append(jax.ShapeDtypeStruct((NC, NS, RPT, 8), jnp.float32))
    scratch += [
        pltpu.VMEM((C, 8), jnp.float32),
        pltpu.VMEM_SHARED((NPAD, 8), jnp.float32),
    ]

  def body(h_hbm, src_hbm, dst_hbm, zacc_hbm, *rest):
    if with_deg:
      (zdeg_hbm, ones_hbm, acc_out, deg_out,
       src_v, dst_v, rows_v, acc_sh, sem, ones_v, deg_sh) = rest
    else:
      (acc_out, src_v, dst_v, rows_v, acc_sh, sem) = rest
    cid = lax.axis_index("c")
    sid = lax.axis_index("s")
    wid = sid * NC + cid

    @pl.when(sid == 0)
    def _():
      pltpu.sync_copy(zacc_hbm, acc_sh)
      if with_deg:
        pltpu.sync_copy(zdeg_hbm, deg_sh)

    if with_deg:
      pltpu.sync_copy(ones_hbm, ones_v)
    plsc.subcore_barrier()

    def step(i, _):
      pltpu.sync_copy(src_hbm.at[wid, i], src_v)
      pltpu.sync_copy(dst_hbm.at[wid, i], dst_v)
      pltpu.async_copy(h_hbm.at[src_v], rows_v, sem).wait()
      pltpu.sync_copy(rows_v, acc_sh.at[dst_v], add=True)
      if with_deg:
        pltpu.sync_copy(ones_v, deg_sh.at[dst_v], add=True)
      return 0

    lax.fori_loop(0, NCH, step, 0)
    plsc.subcore_barrier()
    pltpu.sync_copy(acc_sh.at[pl.ds(sid * RPT, RPT)], acc_out.at[cid, sid])
    if with_deg:
      pltpu.sync_copy(deg_sh.at[pl.ds(sid * RPT, RPT)], deg_out.at[cid, sid])

  return pl.kernel(body, out_type=out_type, mesh=mesh,
                   scratch_types=scratch)


_agg_deg = _make_agg(True)
_agg = _make_agg(False)

BM = 1000


def _tc_body(relu, h_ref, a0_ref, a1_ref, d0_ref, d1_ref, ws_ref, wn_ref,
             b_ref, o_ref):
  deg = d0_ref[...][:, 0:1] + d1_ref[...][:, 0:1]
  hn = (a0_ref[...] + a1_ref[...]) / jnp.maximum(deg, 1.0)
  o = (jnp.dot(h_ref[...], ws_ref[...], preferred_element_type=jnp.float32)
       + jnp.dot(hn, wn_ref[...], preferred_element_type=jnp.float32)
       + b_ref[...])
  o_ref[...] = jnp.maximum(o, 0.0) if relu else o


def _tc_layer(h, a0, a1, d0, d1, W_self, W_neigh, b, relu):
  grid = (N_NODES // BM,)
  row = pl.BlockSpec((BM, D), lambda i: (i, 0))
  deg = pl.BlockSpec((BM, 8), lambda i: (i, 0))
  full = pl.BlockSpec((D, D), lambda i: (0, 0))
  bias = pl.BlockSpec((1, D), lambda i: (0, 0))
  return pl.pallas_call(
      functools.partial(_tc_body, relu),
      grid=grid,
      in_specs=[row, row, row, deg, deg, full, full, bias],
      out_specs=row,
      out_shape=jax.ShapeDtypeStruct((N_NODES, D), jnp.float32),
  )(h, a0, a1, d0, d1, W_self, W_neigh, b.reshape(1, D))


def _pad_edges(a, fill):
  a2 = a.reshape(NW, EPW)
  a2 = jnp.pad(a2, ((0, 0), (0, EPT - EPW)), constant_values=fill)
  return a2.reshape(NW, NCH, C)


def kernel(x, edge_index, W_self1, W_neigh1, b1, W_self2, W_neigh2, b2):
  src = _pad_edges(edge_index[0].astype(jnp.int32), 0)
  dst = _pad_edges(edge_index[1].astype(jnp.int32), N_NODES)
  zacc = jnp.zeros((NPAD, D), jnp.float32)
  zdeg = jnp.zeros((NPAD, 8), jnp.float32)
  ones8 = jnp.zeros((C, 8), jnp.float32).at[:, 0].set(1.0)

  acc1, deg1 = _agg_deg(x, src, dst, zacc, zdeg, ones8)
  a0, a1 = acc1[0].reshape(NPAD, D), acc1[1].reshape(NPAD, D)
  d0, d1 = deg1[0].reshape(NPAD, 8), deg1[1].reshape(NPAD, 8)
  h = _tc_layer(x, a0, a1, d0, d1, W_self1, W_neigh1, b1, relu=True)

  (acc2,) = _agg(h, src, dst, zacc)
  a0, a1 = acc2[0].reshape(NPAD, D), acc2[1].reshape(NPAD, D)
  return _tc_layer(h, a0, a1, d0, d1, W_self2, W_neigh2, b2, relu=False)


# speedup vs baseline: 7.0301x; 7.0301x over previous
import functools

import jax
import jax.numpy as jnp
from jax import lax
from jax.experimental import pallas as pl
from jax.experimental.pallas import tpu as pltpu
from jax.experimental.pallas import tpu_sc as plsc

N_NODES = 10000
D = 128
CW = D // 2
NC = 2
NS = 16
NW = NC * NS
L = 16
E = 320000
C = 128
EPW = E // NS
NCH = -(-EPW // C)
EPT = NCH * C
EPWD = E // NW
NCHD = -(-EPWD // C)
EPTD = NCHD * C
NPAD = 10240
RPT = NPAD // NS
KB = RPT // C

_MESH = plsc.VectorSubcoreMesh(core_axis_name="c", subcore_axis_name="s")


@functools.partial(
    pl.kernel,
    out_type=jax.ShapeDtypeStruct((NC, NPAD, CW), jnp.float32),
    mesh=_MESH,
    scratch_types=[
        pltpu.VMEM((NCH, C), jnp.int32),
        pltpu.VMEM((NCH, C), jnp.int32),
        pltpu.VMEM((C, CW), jnp.float32),
        pltpu.VMEM((C, CW), jnp.float32),
        pltpu.VMEM_SHARED((NPAD, CW), jnp.float32),
        pltpu.SemaphoreType.DMA,
        pltpu.SemaphoreType.DMA,
        pltpu.SemaphoreType.DMA,
    ],
    compiler_params=pltpu.CompilerParams(use_tc_tiling_on_sc=False),
)
def _agg(hh_hbm, src_hbm, dst_hbm, zrows_hbm, acc_out,
         srcs_v, dsts_v, r0, r1, acc_sh, s0, s1, s2):
  rows = (r0, r1)
  sems = (s0, s1)
  cid = lax.axis_index("c")
  sid = lax.axis_index("s")
  base = sid * RPT

  pltpu.sync_copy(src_hbm.at[cid, sid], srcs_v)
  pltpu.sync_copy(dst_hbm.at[sid], dsts_v)
  pltpu.sync_copy(zrows_hbm, r0)
  for k in range(KB):
    pltpu.sync_copy(r0, acc_sh.at[pl.ds(base + k * C, C)])
  plsc.subcore_barrier()

  pltpu.async_copy(hh_hbm.at[srcs_v.at[0]], r0, s0).wait()

  @pl.loop(0, NCH - 1)
  def _(i):
    b = lax.rem(i, 2)

    def issue(b0):
      dsc = pltpu.async_copy(rows[b0], acc_sh.at[dsts_v.at[i]], s2,
                             add=True)
      dg = pltpu.async_copy(hh_hbm.at[srcs_v.at[i + 1]], rows[1 - b0],
                            sems[1 - b0])
      dsc.wait()
      dg.wait()

    @pl.when(b == 0)
    def _():
      issue(0)

    @pl.when(b == 1)
    def _():
      issue(1)

  pltpu.sync_copy(rows[(NCH - 1) % 2], acc_sh.at[dsts_v.at[NCH - 1]],
                  add=True)

  plsc.subcore_barrier()
  for k in range(KB):
    pltpu.sync_copy(acc_sh.at[pl.ds(base + k * C, C)], r0)
    pltpu.sync_copy(r0, acc_out.at[cid, pl.ds(base + k * C, C)])


@functools.partial(
    pl.kernel,
    out_type=jax.ShapeDtypeStruct((NC, NS, NPAD), jnp.float32),
    mesh=_MESH,
    scratch_types=[
        pltpu.VMEM((C,), jnp.int32),
        pltpu.VMEM((NPAD,), jnp.float32),
    ],
    compiler_params=pltpu.CompilerParams(needs_layout_passes=False),
)
def _deg(dst_hbm, zdeg_hbm, deg_out, dst_v, deg_v):
  cid = lax.axis_index("c")
  sid = lax.axis_index("s")
  wid = sid * NC + cid
  pltpu.sync_copy(zdeg_hbm, deg_v)
  one = jnp.ones((L,), jnp.float32)

  @pl.loop(0, NCHD)
  def _(i):
    pltpu.sync_copy(dst_hbm.at[wid, i], dst_v)
    for g in range(C // L):
      idx = dst_v[pl.ds(g * L, L)]
      plsc.addupdate_scatter(deg_v, [idx], one)

  pltpu.sync_copy(deg_v, deg_out.at[cid, sid])


BM = 1000


def _tc1_body(h_ref, a0_ref, a1_ref, dg_ref, ws_ref, wn_ref, b_ref, o_ref):
  deg = jnp.sum(dg_ref[...], axis=1, keepdims=True)
  hn = (jnp.concatenate([a0_ref[...], a1_ref[...]], axis=1)
        / jnp.maximum(deg, 1.0))
  o = (jnp.dot(h_ref[...], ws_ref[...], preferred_element_type=jnp.float32)
       + jnp.dot(hn, wn_ref[...], preferred_element_type=jnp.float32)
       + b_ref[...])
  o = jnp.maximum(o, 0.0)
  o_ref[0] = o[:, :CW]
  o_ref[1] = o[:, CW:]


def _tc_layer1(x, a0, a1, degT, W_self, W_neigh, b):
  grid = (N_NODES // BM,)
  row = pl.BlockSpec((BM, D), lambda i: (i, 0))
  half = pl.BlockSpec((BM, CW), lambda i: (i, 0))
  deg = pl.BlockSpec((BM, NW), lambda i: (i, 0))
  full = pl.BlockSpec((D, D), lambda i: (0, 0))
  bias = pl.BlockSpec((1, D), lambda i: (0, 0))
  return pl.pallas_call(
      _tc1_body,
      grid=grid,
      in_specs=[row, half, half, deg, full, full, bias],
      out_specs=pl.BlockSpec((2, BM, CW), lambda i: (0, i, 0)),
      out_shape=jax.ShapeDtypeStruct((2, N_NODES, CW), jnp.float32),
  )(x, a0, a1, degT, W_self, W_neigh, b.reshape(1, D))


def _tc2_body(hh_ref, a0_ref, a1_ref, dg_ref, ws_ref, wn_ref, b_ref, o_ref):
  deg = jnp.sum(dg_ref[...], axis=1, keepdims=True)
  h = jnp.concatenate([hh_ref[0], hh_ref[1]], axis=1)
  hn = (jnp.concatenate([a0_ref[...], a1_ref[...]], axis=1)
        / jnp.maximum(deg, 1.0))
  o_ref[...] = (
      jnp.dot(h, ws_ref[...], preferred_element_type=jnp.float32)
      + jnp.dot(hn, wn_ref[...], preferred_element_type=jnp.float32)
      + b_ref[...])


def _tc_layer2(hh, a0, a1, degT, W_self, W_neigh, b):
  grid = (N_NODES // BM,)
  row = pl.BlockSpec((BM, D), lambda i: (i, 0))
  half = pl.BlockSpec((BM, CW), lambda i: (i, 0))
  deg = pl.BlockSpec((BM, NW), lambda i: (i, 0))
  full = pl.BlockSpec((D, D), lambda i: (0, 0))
  bias = pl.BlockSpec((1, D), lambda i: (0, 0))
  return pl.pallas_call(
      _tc2_body,
      grid=grid,
      in_specs=[pl.BlockSpec((2, BM, CW), lambda i: (0, i, 0)),
                half, half, deg, full, full, bias],
      out_specs=row,
      out_shape=jax.ShapeDtypeStruct((N_NODES, D), jnp.float32),
  )(hh, a0, a1, degT, W_self, W_neigh, b.reshape(1, D))


def _pad_edges(a, fill, parts, ept):
  epw = E // parts
  a2 = a.reshape(parts, epw)
  a2 = jnp.pad(a2, ((0, 0), (0, ept - epw)), constant_values=fill)
  return a2.reshape(parts, ept // C, C)


def kernel(x, edge_index, W_self1, W_neigh1, b1, W_self2, W_neigh2, b2):
  src_i = edge_index[0].astype(jnp.int32)
  dst_i = edge_index[1].astype(jnp.int32)
  srcA = _pad_edges(src_i, 0, NS, EPT)
  src2 = jnp.stack([srcA, srcA + N_NODES])
  dstA = _pad_edges(dst_i, N_NODES, NS, EPT)
  dstD = _pad_edges(dst_i, N_NODES, NW, EPTD)
  zrows = jnp.zeros((C, CW), jnp.float32)
  zdegv = jnp.zeros((NPAD,), jnp.float32)
  xh = jnp.concatenate([x[:, :CW], x[:, CW:]], axis=0)

  degp = _deg(dstD, zdegv)
  degT = degp.reshape(NW, NPAD).T

  acc1 = _agg(xh, src2, dstA, zrows)
  hh = _tc_layer1(x, acc1[0], acc1[1], degT, W_self1, W_neigh1, b1)

  acc2 = _agg(hh.reshape(NC * N_NODES, CW), src2, dstA, zrows)
  return _tc_layer2(hh, acc2[0], acc2[1], degT, W_self2, W_neigh2, b2)
